# baseline (speedup 1.0000x reference)
import functools

import jax
import jax.numpy as jnp
from jax import lax
from jax.experimental import pallas as pl
from jax.experimental.pallas import tpu as pltpu
from jax.experimental.pallas import tpu_sc as plsc

B = 32
N = 65536
R = 32
V = R * R * R
K = 2048
NCHUNK = N // K
L = 16


def _sc_voxelize(pts):
    mesh = plsc.VectorSubcoreMesh(core_axis_name="c", subcore_axis_name="s")

    @functools.partial(
        pl.kernel,
        mesh=mesh,
        out_type=jax.ShapeDtypeStruct((B, 4, V), jnp.float32),
        compiler_params=pltpu.CompilerParams(
            needs_layout_passes=False, use_tc_tiling_on_sc=False),
        scratch_types=[
            pltpu.VMEM((V,), jnp.float32),
            pltpu.VMEM((V,), jnp.float32),
            pltpu.VMEM((V,), jnp.float32),
            pltpu.VMEM((K, 6), jnp.float32),
            pltpu.VMEM((K, 6), jnp.float32),
            pltpu.SemaphoreType.DMA,
            pltpu.SemaphoreType.DMA,
        ],
    )
    def k(pts_hbm, out_hbm, cnt, hA, hB, buf0, buf1, sem0, sem1):
        b = lax.axis_index("s") * 2 + lax.axis_index("c")

        iota = lax.iota(jnp.int32, L)
        ones = jnp.ones((L,), jnp.float32)
        zeros = jnp.zeros((L,), jnp.float32)
        cols = [jnp.full((L,), c, jnp.int32) for c in range(6)]

        def src(g):
            return pts_hbm.at[b, pl.ds(g * K, K)]

        def stream(proc, init):
            pltpu.make_async_copy(src(0), buf0, sem0).start()
            pltpu.make_async_copy(src(1), buf1, sem1).start()

            def body(t, carry):
                g0 = 2 * t
                pltpu.make_async_copy(src(g0), buf0, sem0).wait()
                carry = proc(buf0, g0, carry)

                @pl.when(g0 + 2 < NCHUNK)
                def _():
                    pltpu.make_async_copy(src(g0 + 2), buf0, sem0).start()

                g1 = g0 + 1
                pltpu.make_async_copy(src(g1), buf1, sem1).wait()
                carry = proc(buf1, g1, carry)

                @pl.when(g1 + 2 < NCHUNK)
                def _():
                    pltpu.make_async_copy(src(g1 + 2), buf1, sem1).start()

                return carry

            return lax.fori_loop(0, NCHUNK // 2, body, init)

        def procS(buf, g, carry):
            def inner(p, carry):
                sx, sy, sz = carry
                ra = iota + p * 32
                rb = ra + L
                sx = sx + plsc.load_gather(buf, [ra, cols[0]])
                sy = sy + plsc.load_gather(buf, [ra, cols[1]])
                sz = sz + plsc.load_gather(buf, [ra, cols[2]])
                sx = sx + plsc.load_gather(buf, [rb, cols[0]])
                sy = sy + plsc.load_gather(buf, [rb, cols[1]])
                sz = sz + plsc.load_gather(buf, [rb, cols[2]])
                return (sx, sy, sz)

            return lax.fori_loop(0, K // (2 * L), inner, carry)

        sx, sy, sz = stream(procS, (zeros, zeros, zeros))
        inv_n = 1.0 / N
        m0 = jnp.full((L,), jnp.sum(sx) * inv_n, jnp.float32)
        m1 = jnp.full((L,), jnp.sum(sy) * inv_n, jnp.float32)
        m2 = jnp.full((L,), jnp.sum(sz) * inv_n, jnp.float32)

        def procM(buf, g, mv):
            def inner(p, mv):
                ra = iota + p * 32
                rb = ra + L
                dxa = plsc.load_gather(buf, [ra, cols[0]]) - m0
                dya = plsc.load_gather(buf, [ra, cols[1]]) - m1
                dza = plsc.load_gather(buf, [ra, cols[2]]) - m2
                mv = jnp.maximum(mv, dxa * dxa + dya * dya + dza * dza)
                dxb = plsc.load_gather(buf, [rb, cols[0]]) - m0
                dyb = plsc.load_gather(buf, [rb, cols[1]]) - m1
                dzb = plsc.load_gather(buf, [rb, cols[2]]) - m2
                return jnp.maximum(mv, dxb * dxb + dyb * dyb + dzb * dzb)

            return lax.fori_loop(0, K // (2 * L), inner, mv)

        maxv = stream(procM, zeros)
        mxv = jnp.full((L,), jnp.max(maxv), jnp.float32)
        iy = jnp.int32(0x5F3759DF) - (
            lax.bitcast_convert_type(mxv, jnp.int32) >> 1)
        y = lax.bitcast_convert_type(iy, jnp.float32)
        for _ in range(4):
            y = y * (1.5 - 0.5 * mxv * y * y)
        sv = 16.0 * y

        def quant(x, m):
            v = (x - m) * sv + 16.0
            v = jnp.minimum(jnp.maximum(v, 0.0), 31.0) + 0.5
            return v.astype(jnp.int32)

        def vox(buf, rows):
            x = plsc.load_gather(buf, [rows, cols[0]])
            y_ = plsc.load_gather(buf, [rows, cols[1]])
            z = plsc.load_gather(buf, [rows, cols[2]])
            return (quant(x, m0) * 32 + quant(y_, m1)) * 32 + quant(z, m2)

        def zero2(ha, hb):
            def zbody(i, _):
                base = i * (8 * L)
                for u in range(8):
                    ha[pl.ds(base + u * L, L)] = zeros
                    hb[pl.ds(base + u * L, L)] = zeros
                return 0

            lax.fori_loop(0, V // (8 * L), zbody, 0)

        zero2(cnt, hA)

        def proc1(buf, g, carry):
            def inner(p, _):
                ra = iota + p * 32
                rb = ra + L
                va = vox(buf, ra)
                vb = vox(buf, rb)
                fa = plsc.load_gather(buf, [ra, cols[3]])
                fb = plsc.load_gather(buf, [rb, cols[3]])
                plsc.addupdate_scatter(cnt, [va], ones)
                plsc.addupdate_scatter(cnt, [vb], ones)
                plsc.addupdate_scatter(hA, [va], fa)
                plsc.addupdate_scatter(hA, [vb], fb)
                return 0

            lax.fori_loop(0, K // (2 * L), inner, 0)
            return carry

        stream(proc1, 0)

        def drain1(i, _):
            s = pl.ds(i * L, L)
            c = cnt[s]
            r = 1.0 / jnp.maximum(c, 1.0)
            hA[s] = hA[s] * r
            return 0

        lax.fori_loop(0, V // L, drain1, 0)
        pltpu.sync_copy(hA, out_hbm.at[b, 1])

        zero2(hA, hB)

        def proc2(buf, g, carry):
            def inner(p, _):
                ra = iota + p * 32
                rb = ra + L
                va = vox(buf, ra)
                vb = vox(buf, rb)
                f4a = plsc.load_gather(buf, [ra, cols[4]])
                f4b = plsc.load_gather(buf, [rb, cols[4]])
                f5a = plsc.load_gather(buf, [ra, cols[5]])
                f5b = plsc.load_gather(buf, [rb, cols[5]])
                plsc.addupdate_scatter(hA, [va], f4a)
                plsc.addupdate_scatter(hA, [vb], f4b)
                plsc.addupdate_scatter(hB, [va], f5a)
                plsc.addupdate_scatter(hB, [vb], f5b)
                return 0

            lax.fori_loop(0, K // (2 * L), inner, 0)
            return carry

        stream(proc2, 0)

        def drain2(i, _):
            s = pl.ds(i * L, L)
            c = cnt[s]
            r = 1.0 / jnp.maximum(c, 1.0)
            hA[s] = hA[s] * r
            hB[s] = hB[s] * r
            cnt[s] = jnp.where(c > 0.0, ones, zeros)
            return 0

        lax.fori_loop(0, V // L, drain2, 0)
        pltpu.sync_copy(cnt, out_hbm.at[b, 0])
        pltpu.sync_copy(hA, out_hbm.at[b, 2])
        pltpu.sync_copy(hB, out_hbm.at[b, 3])

    return k(pts)


def kernel(pts):
    out = _sc_voxelize(pts)
    return out.reshape(B, 4, R, R, R)


# speedup vs baseline: 2.7275x; 2.7275x over previous
import functools

import jax
import jax.numpy as jnp
from jax import lax
from jax.experimental import pallas as pl
from jax.experimental.pallas import tpu as pltpu
from jax.experimental.pallas import tpu_sc as plsc

B = 32
N = 65536
R = 32
V = R * R * R
K = 2048
NCHUNK = N // K
L = 16


def _sc_voxelize(pts):
    mesh = plsc.VectorSubcoreMesh(core_axis_name="c", subcore_axis_name="s")

    @functools.partial(
        pl.kernel,
        mesh=mesh,
        out_type=jax.ShapeDtypeStruct((B, 4, V), jnp.float32),
        compiler_params=pltpu.CompilerParams(
            needs_layout_passes=False, use_tc_tiling_on_sc=False),
        scratch_types=[
            pltpu.VMEM((V,), jnp.float32),
            pltpu.VMEM((V,), jnp.float32),
            pltpu.VMEM((V,), jnp.float32),
            pltpu.VMEM((K * 6,), jnp.float32),
            pltpu.VMEM((K * 6,), jnp.float32),
            pltpu.SemaphoreType.DMA,
            pltpu.SemaphoreType.DMA,
        ],
    )
    def k(pts_hbm, out_hbm, cnt, hA, hB, buf0, buf1, sem0, sem1):
        b = lax.axis_index("s") * 2 + lax.axis_index("c")

        lane6 = lax.iota(jnp.int32, L) * 6
        ones = jnp.ones((L,), jnp.float32)
        zeros = jnp.zeros((L,), jnp.float32)

        def src(g):
            return pts_hbm.at[b, pl.ds(g * K * 6, K * 6)]

        def stream(proc, init):
            pltpu.make_async_copy(src(0), buf0, sem0).start()
            pltpu.make_async_copy(src(1), buf1, sem1).start()

            def body(t, carry):
                g0 = 2 * t
                pltpu.make_async_copy(src(g0), buf0, sem0).wait()
                carry = proc(buf0, g0, carry)

                @pl.when(g0 + 2 < NCHUNK)
                def _():
                    pltpu.make_async_copy(src(g0 + 2), buf0, sem0).start()

                g1 = g0 + 1
                pltpu.make_async_copy(src(g1), buf1, sem1).wait()
                carry = proc(buf1, g1, carry)

                @pl.when(g1 + 2 < NCHUNK)
                def _():
                    pltpu.make_async_copy(src(g1 + 2), buf1, sem1).start()

                return carry

            return lax.fori_loop(0, NCHUNK // 2, body, init)

        def procS(buf, g, carry):
            def inner(p, carry):
                sx, sy, sz = carry
                ixa = lane6 + p * 192
                ixb = ixa + 96
                sx = sx + plsc.load_gather(buf, [ixa])
                sy = sy + plsc.load_gather(buf, [ixa + 1])
                sz = sz + plsc.load_gather(buf, [ixa + 2])
                sx = sx + plsc.load_gather(buf, [ixb])
                sy = sy + plsc.load_gather(buf, [ixb + 1])
                sz = sz + plsc.load_gather(buf, [ixb + 2])
                return (sx, sy, sz)

            return lax.fori_loop(0, K // (2 * L), inner, carry)

        sx, sy, sz = stream(procS, (zeros, zeros, zeros))
        inv_n = 1.0 / N
        m0 = jnp.full((L,), jnp.sum(sx) * inv_n, jnp.float32)
        m1 = jnp.full((L,), jnp.sum(sy) * inv_n, jnp.float32)
        m2 = jnp.full((L,), jnp.sum(sz) * inv_n, jnp.float32)

        def procM(buf, g, mv):
            def inner(p, mv):
                ixa = lane6 + p * 192
                ixb = ixa + 96
                dxa = plsc.load_gather(buf, [ixa]) - m0
                dya = plsc.load_gather(buf, [ixa + 1]) - m1
                dza = plsc.load_gather(buf, [ixa + 2]) - m2
                mv = jnp.maximum(mv, dxa * dxa + dya * dya + dza * dza)
                dxb = plsc.load_gather(buf, [ixb]) - m0
                dyb = plsc.load_gather(buf, [ixb + 1]) - m1
                dzb = plsc.load_gather(buf, [ixb + 2]) - m2
                return jnp.maximum(mv, dxb * dxb + dyb * dyb + dzb * dzb)

            return lax.fori_loop(0, K // (2 * L), inner, mv)

        maxv = stream(procM, zeros)
        mxv = jnp.full((L,), jnp.max(maxv), jnp.float32)
        iy = jnp.int32(0x5F3759DF) - (
            lax.bitcast_convert_type(mxv, jnp.int32) >> 1)
        y = lax.bitcast_convert_type(iy, jnp.float32)
        for _ in range(4):
            y = y * (1.5 - 0.5 * mxv * y * y)
        sv = 16.0 * y

        def quant(x, m):
            v = (x - m) * sv + 16.0
            v = jnp.minimum(jnp.maximum(v, 0.0), 31.0) + 0.5
            return v.astype(jnp.int32)

        def vox(buf, ix):
            x = plsc.load_gather(buf, [ix])
            y_ = plsc.load_gather(buf, [ix + 1])
            z = plsc.load_gather(buf, [ix + 2])
            return (quant(x, m0) * 32 + quant(y_, m1)) * 32 + quant(z, m2)

        def zero2(ha, hb):
            def zbody(i, _):
                base = i * (8 * L)
                for u in range(8):
                    ha[pl.ds(base + u * L, L)] = zeros
                    hb[pl.ds(base + u * L, L)] = zeros
                return 0

            lax.fori_loop(0, V // (8 * L), zbody, 0)

        zero2(cnt, hA)

        def proc1(buf, g, carry):
            def inner(p, _):
                ixa = lane6 + p * 192
                ixb = ixa + 96
                va = vox(buf, ixa)
                vb = vox(buf, ixb)
                fa = plsc.load_gather(buf, [ixa + 3])
                fb = plsc.load_gather(buf, [ixb + 3])
                plsc.addupdate_scatter(cnt, [va], ones)
                plsc.addupdate_scatter(cnt, [vb], ones)
                plsc.addupdate_scatter(hA, [va], fa)
                plsc.addupdate_scatter(hA, [vb], fb)
                return 0

            lax.fori_loop(0, K // (2 * L), inner, 0)
            return carry

        stream(proc1, 0)

        def drain1(i, _):
            s = pl.ds(i * L, L)
            c = cnt[s]
            r = 1.0 / jnp.maximum(c, 1.0)
            hA[s] = hA[s] * r
            return 0

        lax.fori_loop(0, V // L, drain1, 0)
        pltpu.sync_copy(hA, out_hbm.at[b, 1])

        zero2(hA, hB)

        def proc2(buf, g, carry):
            def inner(p, _):
                ixa = lane6 + p * 192
                ixb = ixa + 96
                va = vox(buf, ixa)
                vb = vox(buf, ixb)
                f4a = plsc.load_gather(buf, [ixa + 4])
                f4b = plsc.load_gather(buf, [ixb + 4])
                f5a = plsc.load_gather(buf, [ixa + 5])
                f5b = plsc.load_gather(buf, [ixb + 5])
                plsc.addupdate_scatter(hA, [va], f4a)
                plsc.addupdate_scatter(hA, [vb], f4b)
                plsc.addupdate_scatter(hB, [va], f5a)
                plsc.addupdate_scatter(hB, [vb], f5b)
                return 0

            lax.fori_loop(0, K // (2 * L), inner, 0)
            return carry

        stream(proc2, 0)

        def drain2(i, _):
            s = pl.ds(i * L, L)
            c = cnt[s]
            r = 1.0 / jnp.maximum(c, 1.0)
            hA[s] = hA[s] * r
            hB[s] = hB[s] * r
            cnt[s] = jnp.where(c > 0.0, ones, zeros)
            return 0

        lax.fori_loop(0, V // L, drain2, 0)
        pltpu.sync_copy(cnt, out_hbm.at[b, 0])
        pltpu.sync_copy(hA, out_hbm.at[b, 2])
        pltpu.sync_copy(hB, out_hbm.at[b, 3])

    return k(pts)


def kernel(pts):
    out = _sc_voxelize(pts.reshape(B, N * 6))
    return out.reshape(B, 4, R, R, R)


# speedup vs baseline: 3.0018x; 1.1006x over previous
import functools

import jax
import jax.numpy as jnp
from jax import lax
from jax.experimental import pallas as pl
from jax.experimental.pallas import tpu as pltpu
from jax.experimental.pallas import tpu_sc as plsc

B = 32
N = 65536
R = 32
V = R * R * R
K = 2048
NCHUNK = N // K
L = 16


def _sc_voxelize(pts):
    mesh = plsc.VectorSubcoreMesh(core_axis_name="c", subcore_axis_name="s")

    @functools.partial(
        pl.kernel,
        mesh=mesh,
        out_type=jax.ShapeDtypeStruct((B, 4, V), jnp.float32),
        compiler_params=pltpu.CompilerParams(
            needs_layout_passes=False, use_tc_tiling_on_sc=False),
        scratch_types=[
            pltpu.VMEM((V,), jnp.float32),
            pltpu.VMEM((V,), jnp.float32),
            pltpu.VMEM((V,), jnp.float32),
            pltpu.VMEM((K * 6 // 128, 128), jnp.float32),
            pltpu.VMEM((K * 6 // 128, 128), jnp.float32),
            pltpu.SemaphoreType.DMA,
            pltpu.SemaphoreType.DMA,
        ],
    )
    def k(pts_hbm, out_hbm, cnt, hA, hB, buf0, buf1, sem0, sem1):
        b = lax.axis_index("s") * 2 + lax.axis_index("c")

        lane6 = lax.iota(jnp.int32, L) * 6
        ones = jnp.ones((L,), jnp.float32)
        zeros = jnp.zeros((L,), jnp.float32)

        def src(g):
            return pts_hbm.at[b, pl.ds(g * (K * 6 // 128), K * 6 // 128)]

        def gat(buf, e):
            return plsc.load_gather(buf, [e >> 7, e & 127])

        def stream(proc, init):
            pltpu.make_async_copy(src(0), buf0, sem0).start()
            pltpu.make_async_copy(src(1), buf1, sem1).start()

            def body(t, carry):
                g0 = 2 * t
                pltpu.make_async_copy(src(g0), buf0, sem0).wait()
                carry = proc(buf0, g0, carry)

                @pl.when(g0 + 2 < NCHUNK)
                def _():
                    pltpu.make_async_copy(src(g0 + 2), buf0, sem0).start()

                g1 = g0 + 1
                pltpu.make_async_copy(src(g1), buf1, sem1).wait()
                carry = proc(buf1, g1, carry)

                @pl.when(g1 + 2 < NCHUNK)
                def _():
                    pltpu.make_async_copy(src(g1 + 2), buf1, sem1).start()

                return carry

            return lax.fori_loop(0, NCHUNK // 2, body, init)

        def procS(buf, g, carry):
            def inner(p, carry):
                sx, sy, sz = carry
                ixa = lane6 + p * 192
                ixb = ixa + 96
                sx = sx + gat(buf, ixa)
                sy = sy + gat(buf, ixa + 1)
                sz = sz + gat(buf, ixa + 2)
                sx = sx + gat(buf, ixb)
                sy = sy + gat(buf, ixb + 1)
                sz = sz + gat(buf, ixb + 2)
                return (sx, sy, sz)

            return lax.fori_loop(0, K // (2 * L), inner, carry)

        sx, sy, sz = stream(procS, (zeros, zeros, zeros))
        inv_n = 1.0 / N
        m0 = jnp.full((L,), jnp.sum(sx) * inv_n, jnp.float32)
        m1 = jnp.full((L,), jnp.sum(sy) * inv_n, jnp.float32)
        m2 = jnp.full((L,), jnp.sum(sz) * inv_n, jnp.float32)

        def procM(buf, g, mv):
            def inner(p, mv):
                ixa = lane6 + p * 192
                ixb = ixa + 96
                dxa = gat(buf, ixa) - m0
                dya = gat(buf, ixa + 1) - m1
                dza = gat(buf, ixa + 2) - m2
                mv = jnp.maximum(mv, dxa * dxa + dya * dya + dza * dza)
                dxb = gat(buf, ixb) - m0
                dyb = gat(buf, ixb + 1) - m1
                dzb = gat(buf, ixb + 2) - m2
                return jnp.maximum(mv, dxb * dxb + dyb * dyb + dzb * dzb)

            return lax.fori_loop(0, K // (2 * L), inner, mv)

        maxv = stream(procM, zeros)
        mxv = jnp.full((L,), jnp.max(maxv), jnp.float32)
        iy = jnp.int32(0x5F3759DF) - (
            lax.bitcast_convert_type(mxv, jnp.int32) >> 1)
        y = lax.bitcast_convert_type(iy, jnp.float32)
        for _ in range(4):
            y = y * (1.5 - 0.5 * mxv * y * y)
        sv = 16.0 * y

        def quant(x, m):
            v = (x - m) * sv + 16.0
            v = jnp.minimum(jnp.maximum(v, 0.0), 31.0) + 0.5
            return v.astype(jnp.int32)

        def vox(buf, ix):
            x = gat(buf, ix)
            y_ = gat(buf, ix + 1)
            z = gat(buf, ix + 2)
            return (quant(x, m0) * 32 + quant(y_, m1)) * 32 + quant(z, m2)

        def zero2(ha, hb):
            def zbody(i, _):
                base = i * (8 * L)
                for u in range(8):
                    ha[pl.ds(base + u * L, L)] = zeros
                    hb[pl.ds(base + u * L, L)] = zeros
                return 0

            lax.fori_loop(0, V // (8 * L), zbody, 0)

        zero2(cnt, hA)

        def proc1(buf, g, carry):
            def inner(p, _):
                ixa = lane6 + p * 192
                ixb = ixa + 96
                va = vox(buf, ixa)
                vb = vox(buf, ixb)
                fa = gat(buf, ixa + 3)
                fb = gat(buf, ixb + 3)
                plsc.addupdate_scatter(cnt, [va], ones)
                plsc.addupdate_scatter(cnt, [vb], ones)
                plsc.addupdate_scatter(hA, [va], fa)
                plsc.addupdate_scatter(hA, [vb], fb)
                return 0

            lax.fori_loop(0, K // (2 * L), inner, 0)
            return carry

        stream(proc1, 0)

        def drain1(i, _):
            s = pl.ds(i * L, L)
            c = cnt[s]
            r = 1.0 / jnp.maximum(c, 1.0)
            hA[s] = hA[s] * r
            return 0

        lax.fori_loop(0, V // L, drain1, 0)
        pltpu.sync_copy(hA, out_hbm.at[b, 1])

        zero2(hA, hB)

        def proc2(buf, g, carry):
            def inner(p, _):
                ixa = lane6 + p * 192
                ixb = ixa + 96
                va = vox(buf, ixa)
                vb = vox(buf, ixb)
                f4a = gat(buf, ixa + 4)
                f4b = gat(buf, ixb + 4)
                f5a = gat(buf, ixa + 5)
                f5b = gat(buf, ixb + 5)
                plsc.addupdate_scatter(hA, [va], f4a)
                plsc.addupdate_scatter(hA, [vb], f4b)
                plsc.addupdate_scatter(hB, [va], f5a)
                plsc.addupdate_scatter(hB, [vb], f5b)
                return 0

            lax.fori_loop(0, K // (2 * L), inner, 0)
            return carry

        stream(proc2, 0)

        def drain2(i, _):
            s = pl.ds(i * L, L)
            c = cnt[s]
            r = 1.0 / jnp.maximum(c, 1.0)
            hA[s] = hA[s] * r
            hB[s] = hB[s] * r
            cnt[s] = jnp.where(c > 0.0, ones, zeros)
            return 0

        lax.fori_loop(0, V // L, drain2, 0)
        pltpu.sync_copy(cnt, out_hbm.at[b, 0])
        pltpu.sync_copy(hA, out_hbm.at[b, 2])
        pltpu.sync_copy(hB, out_hbm.at[b, 3])

    return k(pts)


def kernel(pts):
    out = _sc_voxelize(pts.reshape(B, 3072, 128))
    return out.reshape(B, 4, R, R, R)
